# baseline (speedup 1.0000x reference)
import functools

import jax
import jax.numpy as jnp
from jax import lax
from jax.experimental import pallas as pl
from jax.experimental.pallas import tpu as pltpu
from jax.experimental.pallas import tpu_sc as plsc

BATCH = 16384
EMBED = 16
TABLE_ROWS = 1000000
NUM_CORES = 2
NUM_SUBCORES = 16
NUM_WORKERS = NUM_CORES * NUM_SUBCORES
BPW = BATCH // NUM_WORKERS
CHUNK = 128
NCHUNK = BPW // CHUNK


def _sc_gather_dot(uidx2d, pidx2d, uflat, ub_flat, pflat, pb_flat):
    mesh = plsc.VectorSubcoreMesh(core_axis_name="c", subcore_axis_name="s")

    @functools.partial(
        pl.kernel,
        mesh=mesh,
        compiler_params=pltpu.CompilerParams(use_tc_tiling_on_sc=False),
        out_type=[
            jax.ShapeDtypeStruct((NUM_WORKERS, CHUNK), jnp.float32),
            jax.ShapeDtypeStruct((CHUNK, CHUNK), jnp.float32),
        ],
        scratch_types=[
            pltpu.VMEM((NCHUNK, CHUNK), jnp.int32),
            pltpu.VMEM((NCHUNK, CHUNK), jnp.int32),
            pltpu.VMEM((EMBED, NCHUNK, CHUNK), jnp.int32),
            pltpu.VMEM((EMBED, NCHUNK, CHUNK), jnp.int32),
            pltpu.VMEM((EMBED, NCHUNK, CHUNK), jnp.float32),
            pltpu.VMEM((EMBED, NCHUNK, CHUNK), jnp.float32),
            pltpu.VMEM((NCHUNK, CHUNK), jnp.float32),
            pltpu.VMEM((NCHUNK, CHUNK), jnp.float32),
            pltpu.VMEM((CHUNK,), jnp.float32),
            pltpu.SemaphoreType.DMA,
        ],
    )
    def k(uidx_hbm, pidx_hbm, uflat_hbm, ub_hbm, pflat_hbm, pb_hbm,
          part_out, bias_out,
          uidx_v, pidx_v, ufidx_v, pfidx_v, ug_v, pg_v, ub_v, pb_v, acc_v,
          sem):
        wid = lax.axis_index("s") * NUM_CORES + lax.axis_index("c")

        pltpu.sync_copy(uidx_hbm.at[pl.ds(wid * NCHUNK, NCHUNK)], uidx_v)
        pltpu.sync_copy(pidx_hbm.at[pl.ds(wid * NCHUNK, NCHUNK)], pidx_v)

        for c in range(NCHUNK):
            pltpu.async_copy(ub_hbm.at[uidx_v.at[c]], ub_v.at[c], sem)
            pltpu.async_copy(pb_hbm.at[pidx_v.at[c]], pb_v.at[c], sem)

        def fire(j, _):
            off = j * TABLE_ROWS
            for c in range(NCHUNK):
                for s in range(CHUNK // 16):
                    sl = pl.ds(s * 16, 16)
                    ufidx_v[j, c, sl] = uidx_v[c, sl] + off
                    pfidx_v[j, c, sl] = pidx_v[c, sl] + off
                pltpu.async_copy(uflat_hbm.at[ufidx_v.at[j, c]],
                                 ug_v.at[j, c], sem)
                pltpu.async_copy(pflat_hbm.at[pfidx_v.at[j, c]],
                                 pg_v.at[j, c], sem)
            return 0

        lax.fori_loop(0, EMBED, fire, 0)

        for c in range(NCHUNK):
            pltpu.make_async_copy(ub_hbm.at[uidx_v.at[c]], ub_v.at[c],
                                  sem).wait()
            pltpu.make_async_copy(pb_hbm.at[pidx_v.at[c]], pb_v.at[c],
                                  sem).wait()

        def drain(j, _):
            for c in range(NCHUNK):
                pltpu.make_async_copy(uflat_hbm.at[ufidx_v.at[j, c]],
                                      ug_v.at[j, c], sem).wait()
                pltpu.make_async_copy(pflat_hbm.at[pfidx_v.at[j, c]],
                                      pg_v.at[j, c], sem).wait()
            return 0

        lax.fori_loop(0, EMBED, drain, 0)

        for s in range(CHUNK // 16):
            sl = pl.ds(s * 16, 16)
            acc_v[sl] = jnp.zeros((16,), jnp.float32)

        def dot(j, _):
            for c in range(NCHUNK):
                for s in range(CHUNK // 16):
                    sl = pl.ds(s * 16, 16)
                    acc_v[sl] = acc_v[sl] + ug_v[j, c, sl] * pg_v[j, c, sl]
            return 0

        lax.fori_loop(0, EMBED, dot, 0)
        pltpu.sync_copy(acc_v, part_out.at[wid])

        for c in range(NCHUNK):
            for s in range(CHUNK // 16):
                sl = pl.ds(s * 16, 16)
                ub_v[c, sl] = ub_v[c, sl] + pb_v[c, sl]
        pltpu.sync_copy(ub_v, bias_out.at[pl.ds(wid * NCHUNK, NCHUNK)])

    return k(uidx2d, pidx2d, uflat, ub_flat, pflat, pb_flat)


def _tc_finish(part_ref, bias_ref, out_ref):
    s = jnp.sum(part_ref[...])
    out_ref[...] = jax.nn.sigmoid(bias_ref[...] + s)


def kernel(inputs, user_embedding, user_bias, places_embedding, places_bias):
    uidx2d = inputs[:, 0].reshape(NUM_WORKERS * NCHUNK, CHUNK)
    pidx2d = inputs[:, 1].reshape(NUM_WORKERS * NCHUNK, CHUNK)
    partials, bias_sum = _sc_gather_dot(
        uidx2d, pidx2d,
        user_embedding.T.reshape(TABLE_ROWS * EMBED),
        user_bias.reshape(TABLE_ROWS),
        places_embedding.T.reshape(TABLE_ROWS * EMBED),
        places_bias.reshape(TABLE_ROWS))
    out2d = pl.pallas_call(
        _tc_finish,
        out_shape=jax.ShapeDtypeStruct((128, 128), jnp.float32),
    )(partials, bias_sum)
    return out2d.reshape(BATCH, 1)
